# baseline (speedup 1.0000x reference)
import jax
import jax.numpy as jnp
from jax.experimental import pallas as pl
from jax.experimental.pallas import tpu as pltpu

NEG = -1e30
LN2 = 0.6931471805599453
INVLN2 = 1.4426950408889634
_B, _T, _U, _F, _H, _V = 4, 512, 96, 512, 512, 256
UP = 128
UX = 104
VP = 384
TB = 128
BL = _B * UP
K = 8
_CHUNKS = ((0, 32), (32, 32), (64, 32), (96, 8))


def _joint_kernel(nf_ref, frames_ref, wf_ref, ctxoh_ref, e_ref, wo_ref,
                  selaug_ref, blank_ref, lex_ref):
    x = frames_ref[0].astype(jnp.bfloat16)
    wf = wf_ref[...].astype(jnp.bfloat16)
    fproj = jnp.dot(x, wf,
                    preferred_element_type=jnp.float32).astype(jnp.bfloat16)
    cemb = jnp.dot(ctxoh_ref[0], e_ref[...],
                   preferred_element_type=jnp.float32).astype(jnp.bfloat16)
    nfb = nf_ref[pl.program_id(0)]
    t0 = pl.program_id(1) * TB
    tmask = jax.lax.broadcasted_iota(jnp.int32, (TB, UP), 0) + t0 < nfb
    waug = jnp.einsum('uv,hv->uh', selaug_ref[0], wo_ref[...],
                      preferred_element_type=jnp.float32).astype(jnp.bfloat16)
    for u0, uc in _CHUNKS:
        sl = slice(u0, u0 + uc)
        hb = jnp.tanh(fproj[:, None, :] + cemb[None, sl, :])
        zc = jnp.dot(hb.reshape(TB * uc, _H), wo_ref[...],
                     preferred_element_type=jnp.float32)
        z3 = zc.reshape(TB, uc, VP)
        denom = jnp.sum(jnp.exp(z3), axis=-1) - float(VP - (_V + 1))
        lse = jnp.log(denom)
        blankraw = jnp.sum(hb * waug[None, 104:105, :],
                           axis=-1).astype(jnp.float32)
        lexraw = jnp.sum(hb * waug[None, sl, :],
                         axis=-1).astype(jnp.float32)
        tm = tmask[:, sl]
        blank_ref[:, 0, sl] = jnp.where(tm, (blankraw - lse) * INVLN2, 0.0)
        lex_ref[:, 0, sl] = jnp.where(tm, (lexraw - lse) * INVLN2, NEG)
    blank_ref[:, 0, UX:UP] = jnp.zeros((TB, UP - UX), jnp.float32)
    lex_ref[:, 0, UX:UP] = jnp.full((TB, UP - UX), NEG, jnp.float32)


def _roll(x, s):
    return jnp.concatenate([x[:, -s:], x[:, :-s]], axis=1)


def _dp_kernel(nl_ref, blank_ref, lex_ref, out_ref):
    lane = jax.lax.broadcasted_iota(jnp.int32, (1, BL), 1)
    umod = lane & (UP - 1)
    alpha0 = jnp.where(umod == 0, 0.0, jnp.full((1, BL), NEG, jnp.float32))
    seg = lane >> 7
    nl = jnp.where(seg == 0, nl_ref[0],
                   jnp.where(seg == 1, nl_ref[1],
                             jnp.where(seg == 2, nl_ref[2], nl_ref[3])))

    def block(i, r0):
        t0 = i * K
        rs = [r0] + [_roll(r0, s) for s in range(1, K + 1)]
        for j in range(K):
            t = t0 + j
            bt = blank_ref[t]
            lt = lex_ref[t]
            new = []
            for s in range(K - j):
                a = rs[s] + (_roll(bt, s) if s else bt)
                b = rs[s + 1] + _roll(lt, s + 1)
                m = jnp.maximum(a, b)
                new.append(m + jnp.log2(1.0 + jnp.exp2(jnp.minimum(a, b) - m)))
            rs = new
        return rs[0]

    alpha = jax.lax.fori_loop(0, _T // K, block, alpha0)
    sel = jnp.where(umod == nl, alpha, 0.0)
    accs = [jnp.sum(sel[:, b * UP:(b + 1) * UP], axis=1, keepdims=True)
            for b in range(_B)]
    out_ref[...] = jnp.concatenate(accs, axis=1) * (-LN2)


def kernel(frames, num_frames, labels, num_labels, Wf, E, Wo):
    eb = jnp.pad(E.astype(jnp.bfloat16), ((0, VP - (_V + 1)), (0, 0)))
    wob = jnp.pad(Wo.astype(jnp.bfloat16), ((0, 0), (0, VP - (_V + 1))))

    ctx = jnp.concatenate(
        [jnp.zeros((_B, 1), labels.dtype), labels], axis=1)
    ctx_p = jnp.pad(ctx, ((0, 0), (0, UP - (_U + 1))))
    lab_p = jnp.pad(labels, ((0, 0), (0, UP - _U)))
    urow = jnp.arange(UP, dtype=jnp.int32)
    vcol = jnp.arange(VP, dtype=jnp.int32)
    ctxoh = ((ctx_p[:, :, None] == vcol) &
             (urow[None, :, None] <= _U)).astype(jnp.bfloat16)
    selaug = (((lab_p[:, :, None] == vcol) & (urow[None, :, None] < _U)) |
              ((urow[None, :, None] == UX) & (vcol[None, None, :] == 0))
              ).astype(jnp.bfloat16)

    blank, lex = pl.pallas_call(
        _joint_kernel,
        grid=(_B, _T // TB),
        in_specs=[
            pl.BlockSpec(memory_space=pltpu.SMEM),
            pl.BlockSpec((1, TB, _F), lambda b, t: (b, t, 0)),
            pl.BlockSpec((_F, _H), lambda b, t: (0, 0)),
            pl.BlockSpec((1, UP, VP), lambda b, t: (b, 0, 0)),
            pl.BlockSpec((VP, _H), lambda b, t: (0, 0)),
            pl.BlockSpec((_H, VP), lambda b, t: (0, 0)),
            pl.BlockSpec((1, UP, VP), lambda b, t: (b, 0, 0)),
        ],
        out_specs=[
            pl.BlockSpec((TB, 1, UP), lambda b, t: (t, 0, b)),
            pl.BlockSpec((TB, 1, UP), lambda b, t: (t, 0, b)),
        ],
        out_shape=[
            jax.ShapeDtypeStruct((_T, 1, BL), jnp.float32),
            jax.ShapeDtypeStruct((_T, 1, BL), jnp.float32),
        ],
        compiler_params=pltpu.CompilerParams(
            dimension_semantics=("parallel", "arbitrary"),
            allow_input_fusion=[False, False, True, True, True, True, True],
        ),
        name="lattice_joint",
    )(num_frames, frames, Wf, ctxoh, eb, wob, selaug)

    out = pl.pallas_call(
        _dp_kernel,
        in_specs=[
            pl.BlockSpec(memory_space=pltpu.SMEM),
            pl.BlockSpec(memory_space=pltpu.VMEM),
            pl.BlockSpec(memory_space=pltpu.VMEM),
        ],
        out_shape=jax.ShapeDtypeStruct((1, _B), jnp.float32),
        name="lattice_dp",
    )(num_labels, blank, lex)
    return out.reshape(_B)


# speedup vs baseline: 1.1700x; 1.1700x over previous
import jax
import jax.numpy as jnp
from jax.experimental import pallas as pl
from jax.experimental.pallas import tpu as pltpu

NEG = -1e30
LN2 = 0.6931471805599453
INVLN2 = 1.4426950408889634
_B, _T, _U, _F, _H, _V = 4, 512, 96, 512, 512, 256
UP = 128
UX = 104
VP = 384
TB = 128
BL = _B * UP
K = 8
_CHUNKS = ((0, 32), (32, 32), (64, 32), (96, 8))


def _joint_kernel(nf_ref, frames_ref, wf_ref, ctxoh_ref, e_ref, wo_ref,
                  lexoh_ref, blank_ref, lex_ref):
    x = frames_ref[0].astype(jnp.bfloat16)
    wf = wf_ref[...].astype(jnp.bfloat16)
    fproj = jnp.dot(x, wf,
                    preferred_element_type=jnp.float32).astype(jnp.bfloat16)
    cemb = jnp.dot(ctxoh_ref[0], e_ref[...],
                   preferred_element_type=jnp.float32).astype(jnp.bfloat16)
    vlane = jax.lax.broadcasted_iota(jnp.int32, (1, 1, VP), 2)
    nfb = nf_ref[pl.program_id(0)]
    t0 = pl.program_id(1) * TB
    tmask = jax.lax.broadcasted_iota(jnp.int32, (TB, UP), 0) + t0 < nfb
    for u0, uc in _CHUNKS:
        sl = slice(u0, u0 + uc)
        hb = jnp.tanh(fproj[:, None, :] + cemb[None, sl, :])
        zc = jnp.dot(hb.reshape(TB * uc, _H), wo_ref[...],
                     preferred_element_type=jnp.float32)
        z3 = zc.reshape(TB, uc, VP)
        denom = jnp.sum(jnp.exp2(z3), axis=-1) - float(VP - (_V + 1))
        lse = jnp.log2(denom)
        blankraw = jnp.sum(jnp.where(vlane == 0, z3, 0.0), axis=-1)
        lexraw = jnp.sum(z3 * lexoh_ref[0][None, sl, :], axis=-1)
        tm = tmask[:, sl]
        blank_ref[:, 0, sl] = jnp.where(tm, blankraw - lse, 0.0)
        lex_ref[:, 0, sl] = jnp.where(tm, lexraw - lse, NEG)
    blank_ref[:, 0, UX:UP] = jnp.zeros((TB, UP - UX), jnp.float32)
    lex_ref[:, 0, UX:UP] = jnp.full((TB, UP - UX), NEG, jnp.float32)


def _roll(x, s):
    return jnp.concatenate([x[:, -s:], x[:, :-s]], axis=1)


def _dp_kernel(nl_ref, blank_ref, lex_ref, out_ref):
    lane = jax.lax.broadcasted_iota(jnp.int32, (1, BL), 1)
    umod = lane & (UP - 1)
    alpha0 = jnp.where(umod == 0, 0.0, jnp.full((1, BL), NEG, jnp.float32))
    seg = lane >> 7
    nl = jnp.where(seg == 0, nl_ref[0],
                   jnp.where(seg == 1, nl_ref[1],
                             jnp.where(seg == 2, nl_ref[2], nl_ref[3])))

    def block(i, r0):
        t0 = i * K
        rs = [r0] + [_roll(r0, s) for s in range(1, K + 1)]
        for j in range(K):
            t = t0 + j
            bt = blank_ref[t]
            lt = lex_ref[t]
            new = []
            for s in range(K - j):
                a = rs[s] + (_roll(bt, s) if s else bt)
                b = rs[s + 1] + _roll(lt, s + 1)
                m = jnp.maximum(a, b)
                new.append(m + jnp.log2(1.0 + jnp.exp2(jnp.minimum(a, b) - m)))
            rs = new
        return rs[0]

    alpha = jax.lax.fori_loop(0, _T // K, block, alpha0)
    sel = jnp.where(umod == nl, alpha, 0.0)
    accs = [jnp.sum(sel[:, b * UP:(b + 1) * UP], axis=1, keepdims=True)
            for b in range(_B)]
    out_ref[...] = jnp.concatenate(accs, axis=1) * (-LN2)


def kernel(frames, num_frames, labels, num_labels, Wf, E, Wo):
    eb = jnp.pad(E.astype(jnp.bfloat16), ((0, VP - (_V + 1)), (0, 0)))
    wob = jnp.pad((Wo * INVLN2).astype(jnp.bfloat16),
                  ((0, 0), (0, VP - (_V + 1))))

    ctx = jnp.concatenate(
        [jnp.zeros((_B, 1), labels.dtype), labels], axis=1)
    ctx_p = jnp.pad(ctx, ((0, 0), (0, UP - (_U + 1))))
    lab_p = jnp.pad(labels, ((0, 0), (0, UP - _U)))
    urow = jnp.arange(UP, dtype=jnp.int32)
    vcol = jnp.arange(VP, dtype=jnp.int32)
    ctxoh = ((ctx_p[:, :, None] == vcol) &
             (urow[None, :, None] <= _U)).astype(jnp.bfloat16)
    lexoh = ((lab_p[:, :, None] == vcol) &
             (urow[None, :, None] < _U)).astype(jnp.float32)

    blank, lex = pl.pallas_call(
        _joint_kernel,
        grid=(_B, _T // TB),
        in_specs=[
            pl.BlockSpec(memory_space=pltpu.SMEM),
            pl.BlockSpec((1, TB, _F), lambda b, t: (b, t, 0)),
            pl.BlockSpec((_F, _H), lambda b, t: (0, 0)),
            pl.BlockSpec((1, UP, VP), lambda b, t: (b, 0, 0)),
            pl.BlockSpec((VP, _H), lambda b, t: (0, 0)),
            pl.BlockSpec((_H, VP), lambda b, t: (0, 0)),
            pl.BlockSpec((1, UP, VP), lambda b, t: (b, 0, 0)),
        ],
        out_specs=[
            pl.BlockSpec((TB, 1, UP), lambda b, t: (t, 0, b)),
            pl.BlockSpec((TB, 1, UP), lambda b, t: (t, 0, b)),
        ],
        out_shape=[
            jax.ShapeDtypeStruct((_T, 1, BL), jnp.float32),
            jax.ShapeDtypeStruct((_T, 1, BL), jnp.float32),
        ],
        compiler_params=pltpu.CompilerParams(
            dimension_semantics=("parallel", "arbitrary"),
            allow_input_fusion=[False, False, True, True, True, True, True],
        ),
        name="lattice_joint",
    )(num_frames, frames, Wf, ctxoh, eb, wob, lexoh)

    out = pl.pallas_call(
        _dp_kernel,
        in_specs=[
            pl.BlockSpec(memory_space=pltpu.SMEM),
            pl.BlockSpec(memory_space=pltpu.VMEM),
            pl.BlockSpec(memory_space=pltpu.VMEM),
        ],
        out_shape=jax.ShapeDtypeStruct((1, _B), jnp.float32),
        name="lattice_dp",
    )(num_labels, blank, lex)
    return out.reshape(_B)


# speedup vs baseline: 1.1923x; 1.0190x over previous
import jax
import jax.numpy as jnp
from jax.experimental import pallas as pl
from jax.experimental.pallas import tpu as pltpu

NEG = -1e30
LN2 = 0.6931471805599453
INVLN2 = 1.4426950408889634
_B, _T, _U, _F, _H, _V = 4, 512, 96, 512, 512, 256
UP = 128
UX = 104
VP = 384
TB = 256
BL = _B * UP
K = 8
_CHUNKS = ((0, 32), (32, 32), (64, 32), (96, 8))


def _joint_kernel(nf_ref, frames_ref, wf_ref, ctxoh_ref, e_ref, wo_ref,
                  lexoh_ref, blank_ref, lex_ref):
    x = frames_ref[0].astype(jnp.bfloat16)
    wf = wf_ref[...].astype(jnp.bfloat16)
    fproj = jnp.dot(x, wf,
                    preferred_element_type=jnp.float32).astype(jnp.bfloat16)
    cemb = jnp.dot(ctxoh_ref[0], e_ref[...],
                   preferred_element_type=jnp.float32).astype(jnp.bfloat16)
    vlane = jax.lax.broadcasted_iota(jnp.int32, (1, 1, VP), 2)
    nfb = nf_ref[pl.program_id(0)]
    t0 = pl.program_id(1) * TB
    tmask = jax.lax.broadcasted_iota(jnp.int32, (TB, UP), 0) + t0 < nfb
    for u0, uc in _CHUNKS:
        sl = slice(u0, u0 + uc)
        hb = jnp.tanh(fproj[:, None, :] + cemb[None, sl, :])
        zc = jnp.dot(hb.reshape(TB * uc, _H), wo_ref[...],
                     preferred_element_type=jnp.float32)
        z3 = zc.reshape(TB, uc, VP)
        denom = jnp.sum(jnp.exp2(z3), axis=-1) - float(VP - (_V + 1))
        lse = jnp.log2(denom)
        blankraw = jnp.sum(jnp.where(vlane == 0, z3, 0.0), axis=-1)
        lexraw = jnp.sum(z3 * lexoh_ref[0][None, sl, :], axis=-1)
        tm = tmask[:, sl]
        blank_ref[:, 0, sl] = jnp.where(tm, blankraw - lse, 0.0)
        lex_ref[:, 0, sl] = jnp.where(tm, lexraw - lse, NEG)
    blank_ref[:, 0, UX:UP] = jnp.zeros((TB, UP - UX), jnp.float32)
    lex_ref[:, 0, UX:UP] = jnp.full((TB, UP - UX), NEG, jnp.float32)


def _roll(x, s):
    return jnp.concatenate([x[:, -s:], x[:, :-s]], axis=1)


def _dp_kernel(nl_ref, blank_ref, lex_ref, out_ref):
    lane = jax.lax.broadcasted_iota(jnp.int32, (1, BL), 1)
    umod = lane & (UP - 1)
    alpha0 = jnp.where(umod == 0, 0.0, jnp.full((1, BL), NEG, jnp.float32))
    seg = lane >> 7
    nl = jnp.where(seg == 0, nl_ref[0],
                   jnp.where(seg == 1, nl_ref[1],
                             jnp.where(seg == 2, nl_ref[2], nl_ref[3])))

    def block(i, r0):
        t0 = i * K
        rs = [r0] + [_roll(r0, s) for s in range(1, K + 1)]
        for j in range(K):
            t = t0 + j
            bt = blank_ref[t]
            lt = lex_ref[t]
            new = []
            for s in range(K - j):
                a = rs[s] + (_roll(bt, s) if s else bt)
                b = rs[s + 1] + _roll(lt, s + 1)
                m = jnp.maximum(a, b)
                new.append(m + jnp.log2(1.0 + jnp.exp2(jnp.minimum(a, b) - m)))
            rs = new
        return rs[0]

    alpha = jax.lax.fori_loop(0, _T // K, block, alpha0)
    sel = jnp.where(umod == nl, alpha, 0.0)
    accs = [jnp.sum(sel[:, b * UP:(b + 1) * UP], axis=1, keepdims=True)
            for b in range(_B)]
    out_ref[...] = jnp.concatenate(accs, axis=1) * (-LN2)


def kernel(frames, num_frames, labels, num_labels, Wf, E, Wo):
    eb = jnp.pad(E.astype(jnp.bfloat16), ((0, VP - (_V + 1)), (0, 0)))
    wob = jnp.pad((Wo * INVLN2).astype(jnp.bfloat16),
                  ((0, 0), (0, VP - (_V + 1))))

    ctx = jnp.concatenate(
        [jnp.zeros((_B, 1), labels.dtype), labels], axis=1)
    ctx_p = jnp.pad(ctx, ((0, 0), (0, UP - (_U + 1))))
    lab_p = jnp.pad(labels, ((0, 0), (0, UP - _U)))
    urow = jnp.arange(UP, dtype=jnp.int32)
    vcol = jnp.arange(VP, dtype=jnp.int32)
    ctxoh = ((ctx_p[:, :, None] == vcol) &
             (urow[None, :, None] <= _U)).astype(jnp.bfloat16)
    lexoh = ((lab_p[:, :, None] == vcol) &
             (urow[None, :, None] < _U)).astype(jnp.float32)

    blank, lex = pl.pallas_call(
        _joint_kernel,
        grid=(_B, _T // TB),
        in_specs=[
            pl.BlockSpec(memory_space=pltpu.SMEM),
            pl.BlockSpec((1, TB, _F), lambda b, t: (b, t, 0)),
            pl.BlockSpec((_F, _H), lambda b, t: (0, 0)),
            pl.BlockSpec((1, UP, VP), lambda b, t: (b, 0, 0)),
            pl.BlockSpec((VP, _H), lambda b, t: (0, 0)),
            pl.BlockSpec((_H, VP), lambda b, t: (0, 0)),
            pl.BlockSpec((1, UP, VP), lambda b, t: (b, 0, 0)),
        ],
        out_specs=[
            pl.BlockSpec((TB, 1, UP), lambda b, t: (t, 0, b)),
            pl.BlockSpec((TB, 1, UP), lambda b, t: (t, 0, b)),
        ],
        out_shape=[
            jax.ShapeDtypeStruct((_T, 1, BL), jnp.float32),
            jax.ShapeDtypeStruct((_T, 1, BL), jnp.float32),
        ],
        compiler_params=pltpu.CompilerParams(
            dimension_semantics=("parallel", "arbitrary"),
            allow_input_fusion=[False, False, True, True, True, True, True],
            vmem_limit_bytes=50 * 1024 * 1024,
        ),
        name="lattice_joint",
    )(num_frames, frames, Wf, ctxoh, eb, wob, lexoh)

    out = pl.pallas_call(
        _dp_kernel,
        in_specs=[
            pl.BlockSpec(memory_space=pltpu.SMEM),
            pl.BlockSpec(memory_space=pltpu.VMEM),
            pl.BlockSpec(memory_space=pltpu.VMEM),
        ],
        out_shape=jax.ShapeDtypeStruct((1, _B), jnp.float32),
        name="lattice_dp",
    )(num_labels, blank, lex)
    return out.reshape(_B)
